# baseline (speedup 1.0000x reference)
import functools

import jax
import jax.numpy as jnp
from jax import lax
from jax.experimental import pallas as pl
from jax.experimental.pallas import tpu as pltpu
from jax.experimental.pallas import tpu_sc as plsc

_B = 8
_N = 2048
_EMB = 128
_MAXD = 25

_ROW_BLK = 256

_NC = 2
_NS = 16
_NW = _NC * _NS
_TOTAL = _B * _N
_PER_W = _TOTAL // _NW
_IDX_MINOR = 128
_IDX_ROWS_PER_W = _PER_W // _IDX_MINOR


def _deg_body(adj_ref, deg_ref):
    s = jnp.sum(adj_ref[...], axis=-1)
    deg_ref[...] = jnp.minimum(
        jnp.round(s), float(_MAXD)).astype(jnp.int32).reshape(1, 1, _ROW_BLK)


def _degrees(adj):
    nblk = _B * _N // _ROW_BLK
    adj3 = adj.reshape(nblk, _ROW_BLK, _N)
    return pl.pallas_call(
        _deg_body,
        grid=(nblk,),
        in_specs=[pl.BlockSpec((1, _ROW_BLK, _N), lambda i: (i, 0, 0))],
        out_specs=pl.BlockSpec((1, 1, _ROW_BLK), lambda i: (i, 0, 0)),
        out_shape=jax.ShapeDtypeStruct((nblk, 1, _ROW_BLK), jnp.int32),
        compiler_params=pltpu.CompilerParams(
            dimension_semantics=("arbitrary",),
        ),
    )(adj3)


@functools.lru_cache(maxsize=1)
def _sc_gather_fn():
    mesh = plsc.VectorSubcoreMesh(core_axis_name="c", subcore_axis_name="s")

    @functools.partial(
        pl.kernel,
        mesh=mesh,
        out_type=jax.ShapeDtypeStruct((_TOTAL, _EMB), jnp.float32),
        scratch_types=[
            pltpu.VMEM((_IDX_ROWS_PER_W, _IDX_MINOR), jnp.int32),
            pltpu.VMEM((_PER_W, _EMB), jnp.float32),
            pltpu.SemaphoreType.DMA,
        ],
    )
    def _sc_gather(table_hbm, idx_hbm, out_hbm, idx_v, rows_v, sem):
        wid = lax.axis_index("s") * _NC + lax.axis_index("c")
        pltpu.sync_copy(
            idx_hbm.at[pl.ds(wid * _IDX_ROWS_PER_W, _IDX_ROWS_PER_W)], idx_v)
        copies = [
            pltpu.async_copy(
                table_hbm.at[idx_v.at[j]],
                rows_v.at[pl.ds(j * _IDX_MINOR, _IDX_MINOR)],
                sem,
            )
            for j in range(_IDX_ROWS_PER_W)
        ]
        for c in copies:
            c.wait()
        pltpu.sync_copy(rows_v, out_hbm.at[pl.ds(wid * _PER_W, _PER_W)])

    return _sc_gather


def kernel(data, adj, dense, emb_weight):
    int_deg = _degrees(adj)
    idx2d = int_deg.reshape(_TOTAL // _IDX_MINOR, _IDX_MINOR)
    rows = _sc_gather_fn()(emb_weight, idx2d)
    return rows.reshape(_B, _N, _EMB)
